# baseline (speedup 1.0000x reference)
import functools

import jax
import jax.numpy as jnp
from jax import lax
from jax.experimental import pallas as pl
from jax.experimental.pallas import tpu as pltpu
from jax.experimental.pallas import tpu_sc as plsc

BATCH = 16384
DIM = 16
NC = 2
NS = 16
NW = NC * NS
B_PER_W = BATCH // NW
CHUNK = 128
NCHUNK = B_PER_W // CHUNK


def _sc_body(users_hbm, items_hbm, utab_hbm, itab_hbm, out_hbm,
             idx_v, ubuf_v, ibuf_v, out_v, sem):
    wid = lax.axis_index("s") * NC + lax.axis_index("c")
    base = wid * B_PER_W

    for c in range(NCHUNK):
        pltpu.sync_copy(users_hbm.at[pl.ds(base + c * CHUNK, CHUNK)],
                        idx_v.at[c])
        pltpu.sync_copy(items_hbm.at[pl.ds(base + c * CHUNK, CHUNK)],
                        idx_v.at[NCHUNK + c])

    def fire_d(d, carry):
        for c in range(NCHUNK):
            pltpu.async_copy(
                utab_hbm.at[d].at[idx_v.at[c]],
                ubuf_v.at[d, pl.ds(c * CHUNK, CHUNK)], sem)
            pltpu.async_copy(
                itab_hbm.at[d].at[idx_v.at[NCHUNK + c]],
                ibuf_v.at[d, pl.ds(c * CHUNK, CHUNK)], sem)
        return carry

    lax.fori_loop(0, DIM, fire_d, 0, unroll=False)

    pltpu.make_async_copy(utab_hbm.at[pl.ds(0, DIM), pl.ds(0, B_PER_W)],
                          ubuf_v, sem).wait()
    pltpu.make_async_copy(itab_hbm.at[pl.ds(0, DIM), pl.ds(0, B_PER_W)],
                          ibuf_v, sem).wait()

    def block_body(b, carry):
        sl = pl.ds(b * 16, 16)

        def dot_d(d, acc):
            return acc + ubuf_v[d, sl] * ibuf_v[d, sl]

        acc = lax.fori_loop(0, DIM, dot_d, jnp.zeros((16,), jnp.float32),
                            unroll=False)
        out_v[sl] = 1.0 / (1.0 + jnp.exp(-acc))
        return carry

    lax.fori_loop(0, B_PER_W // 16, block_body, 0, unroll=False)

    pltpu.sync_copy(out_v, out_hbm.at[pl.ds(base, B_PER_W)])


def kernel(users, items, group, group_items, user_table, item_table,
           group_table, group_item_table):
    utab = user_table.T
    itab = item_table.T
    mesh = plsc.VectorSubcoreMesh(core_axis_name="c", subcore_axis_name="s")
    run = functools.partial(
        pl.kernel,
        mesh=mesh,
        compiler_params=pltpu.CompilerParams(
            needs_layout_passes=False, use_tc_tiling_on_sc=False),
        out_type=jax.ShapeDtypeStruct((BATCH,), jnp.float32),
        scratch_types=[
            pltpu.VMEM((2 * NCHUNK, CHUNK), jnp.int32),
            pltpu.VMEM((DIM, B_PER_W), jnp.float32),
            pltpu.VMEM((DIM, B_PER_W), jnp.float32),
            pltpu.VMEM((B_PER_W,), jnp.float32),
            pltpu.SemaphoreType.DMA,
        ],
    )(_sc_body)
    return run(users, items, utab, itab)


# speedup vs baseline: 55.1569x; 55.1569x over previous
import functools

import jax
import jax.numpy as jnp
from jax import lax
from jax.experimental import pallas as pl
from jax.experimental.pallas import tpu as pltpu
from jax.experimental.pallas import tpu_sc as plsc

BATCH = 16384
DIM = 16
HD = 8
W = 16
NC = 2
NS = 16
NW = NC * NS
B_PER_W = BATCH // NW
CH = 16
NCHUNK = B_PER_W // CH


def _sc_body(users_hbm, items_hbm, utab_hbm, itab_hbm, out_hbm,
             idx_v, sub_v, ua_v, ub_v, ia_v, ib_v, out_v, sema, semb):
    wid = lax.axis_index("s") * NC + lax.axis_index("c")
    base = wid * B_PER_W

    pltpu.sync_copy(users_hbm.at[pl.ds(base, B_PER_W)], idx_v.at[0])
    pltpu.sync_copy(items_hbm.at[pl.ds(base, B_PER_W)], idx_v.at[1])

    lane = lax.iota(jnp.int32, 16)

    def prep_body(k, carry):
        for t in range(2):
            raw = idx_v[t, pl.ds(k * 16, 16)]
            idx_v[t, pl.ds(k * 16, 16)] = lax.bitwise_and(raw, ~15)
            sub_v[t, pl.ds(k * 16, 16)] = lax.bitwise_and(raw, 15)
        return carry

    lax.fori_loop(0, B_PER_W // 16, prep_body, 0, unroll=False)

    def fire(c, urows, irows, sem):
        uvec = idx_v[0, pl.ds(c * CH, 16)]
        ivec = idx_v[1, pl.ds(c * CH, 16)]

        def fire_j(j, carry):
            m = lane == j
            uc = pl.multiple_of(jnp.sum(jnp.where(m, uvec, 0)), W)
            ic = pl.multiple_of(jnp.sum(jnp.where(m, ivec, 0)), W)
            dst = pl.ds(j * W, W)
            for h in range(2):
                pltpu.async_copy(
                    utab_hbm.at[pl.ds(h * HD, HD), pl.ds(uc, W)],
                    urows.at[pl.ds(h * HD, HD), dst], sem)
                pltpu.async_copy(
                    itab_hbm.at[pl.ds(h * HD, HD), pl.ds(ic, W)],
                    irows.at[pl.ds(h * HD, HD), dst], sem)
            return carry

        lax.fori_loop(0, 16, fire_j, 0, unroll=False)

    def drain(urows, irows, sem):
        pltpu.make_async_copy(utab_hbm.at[pl.ds(0, DIM), pl.ds(0, CH * W)],
                              urows, sem).wait()
        pltpu.make_async_copy(itab_hbm.at[pl.ds(0, DIM), pl.ds(0, CH * W)],
                              irows, sem).wait()

    def compute(c, urows, irows):
        ucols = lane * W + sub_v[0, pl.ds(c * CH, 16)]
        icols = lane * W + sub_v[1, pl.ds(c * CH, 16)]

        def dot_d(d, acc):
            dvec = jnp.full((16,), d, jnp.int32)
            uu = plsc.load_gather(urows, [dvec, ucols])
            vv = plsc.load_gather(irows, [dvec, icols])
            return acc + uu * vv

        acc = lax.fori_loop(0, DIM, dot_d, jnp.zeros((16,), jnp.float32),
                            unroll=False)
        out_v[pl.ds(c * CH, 16)] = 1.0 / (1.0 + jnp.exp(-acc))

    fire(0, ua_v, ia_v, sema)

    def pair_body(k, carry):
        ca = 2 * k
        fire(ca + 1, ub_v, ib_v, semb)
        drain(ua_v, ia_v, sema)
        compute(ca, ua_v, ia_v)
        fire(ca + 2, ua_v, ia_v, sema)
        drain(ub_v, ib_v, semb)
        compute(ca + 1, ub_v, ib_v)
        return carry

    lax.fori_loop(0, NCHUNK // 2 - 1, pair_body, 0, unroll=False)

    fire(NCHUNK - 1, ub_v, ib_v, semb)
    drain(ua_v, ia_v, sema)
    compute(NCHUNK - 2, ua_v, ia_v)
    drain(ub_v, ib_v, semb)
    compute(NCHUNK - 1, ub_v, ib_v)

    pltpu.sync_copy(out_v, out_hbm.at[pl.ds(base, B_PER_W)])


def kernel(users, items, group, group_items, user_table, item_table,
           group_table, group_item_table):
    utab = user_table.T
    itab = item_table.T
    mesh = plsc.VectorSubcoreMesh(core_axis_name="c", subcore_axis_name="s")
    run = functools.partial(
        pl.kernel,
        mesh=mesh,
        compiler_params=pltpu.CompilerParams(
            needs_layout_passes=False, use_tc_tiling_on_sc=True),
        out_type=jax.ShapeDtypeStruct((BATCH,), jnp.float32),
        scratch_types=[
            pltpu.VMEM((2, B_PER_W), jnp.int32),
            pltpu.VMEM((2, B_PER_W), jnp.int32),
            pltpu.VMEM((DIM, CH * W), jnp.float32),
            pltpu.VMEM((DIM, CH * W), jnp.float32),
            pltpu.VMEM((DIM, CH * W), jnp.float32),
            pltpu.VMEM((DIM, CH * W), jnp.float32),
            pltpu.VMEM((B_PER_W,), jnp.float32),
            pltpu.SemaphoreType.DMA,
            pltpu.SemaphoreType.DMA,
        ],
    )(_sc_body)
    return run(users, items, utab, itab)
